# Initial kernel scaffold; baseline (speedup 1.0000x reference)
import jax
import jax.numpy as jnp
from jax.experimental import pallas as pl


def kernel(atom_types, pos):
    raise NotImplementedError("write your pallas kernel here")


# speedup vs baseline: 1.8730x; 1.8730x over previous
import functools

import jax
import jax.numpy as jnp
from jax import lax
from jax.experimental import pallas as pl
from jax.experimental.pallas import tpu as pltpu
from jax.experimental.pallas import tpu_sc as plsc

_NUM_TYPES = 64
_N = 100000
_LANES = 16
_GROUPS = _N // _LANES
_NC, _NS = 2, 16
_NW = _NC * _NS
_BASE_G = _GROUPS // _NW
_EXTRA = _GROUPS % _NW
_CG = 40
_CHUNK_ATOMS = _CG * _LANES
_CHUNK_WORDS = _CHUNK_ATOMS * _NUM_TYPES


def _one_hot_sc(types_flat):
    mesh = plsc.VectorSubcoreMesh(
        core_axis_name="c", subcore_axis_name="s",
        num_cores=_NC, num_subcores=_NS,
    )

    @functools.partial(
        pl.kernel,
        mesh=mesh,
        out_type=(
            jax.ShapeDtypeStruct((_N * _NUM_TYPES,), jnp.float32),
            jax.ShapeDtypeStruct((_N * _NUM_TYPES,), jnp.float32),
        ),
        scratch_types=[
            pltpu.VMEM((_CHUNK_ATOMS,), jnp.int32),
            pltpu.VMEM((_CHUNK_WORDS,), jnp.float32),
        ],
        compiler_params=pltpu.CompilerParams(needs_layout_passes=False),
    )
    def k(types_hbm, out_a_hbm, out_b_hbm, idx_v, rows_v):
        wid = lax.axis_index("s") * _NC + lax.axis_index("c")
        g0 = wid * _BASE_G + jnp.minimum(wid, _EXTRA)
        cnt = _BASE_G + jnp.where(wid < _EXTRA, 1, 0)

        lane_row = lax.iota(jnp.int32, _LANES) * _NUM_TYPES
        ones = jnp.full((_LANES,), 1.0, jnp.float32)
        zeros = jnp.zeros((_LANES,), jnp.float32)

        def zbody(i, carry):
            base = i * (16 * _LANES)
            for u in range(16):
                rows_v[pl.ds(base + u * _LANES, _LANES)] = zeros
            return carry

        lax.fori_loop(0, _CHUNK_WORDS // (16 * _LANES), zbody, 0)

        def do_chunk(g, clear):
            pltpu.sync_copy(types_hbm.at[pl.ds(g * _LANES, _CHUNK_ATOMS)],
                            idx_v)

            def sbody(j, carry):
                t = idx_v[pl.ds(j * _LANES, _LANES)]
                fidx = lane_row + t + j * (_LANES * _NUM_TYPES)
                plsc.store_scatter(rows_v, [fidx], ones)
                return carry

            lax.fori_loop(0, _CG, sbody, 0)

            base = g * (_LANES * _NUM_TYPES)
            pltpu.sync_copy(rows_v, out_a_hbm.at[pl.ds(base, _CHUNK_WORDS)])
            pltpu.sync_copy(rows_v, out_b_hbm.at[pl.ds(base, _CHUNK_WORDS)])

            if clear:
                def cbody(j, carry):
                    t = idx_v[pl.ds(j * _LANES, _LANES)]
                    fidx = lane_row + t + j * (_LANES * _NUM_TYPES)
                    plsc.store_scatter(rows_v, [fidx], zeros)
                    return carry

                lax.fori_loop(0, _CG, cbody, 0)

        for c in range(4):
            do_chunk(g0 + c * _CG, clear=True)
        do_chunk(g0 + cnt - _CG, clear=False)

    return k(types_flat)


def kernel(atom_types, pos):
    out_a, out_b = _one_hot_sc(atom_types.reshape(-1))
    out_a = out_a.reshape(_N, _NUM_TYPES).astype(pos.dtype)
    out_b = out_b.reshape(_N, _NUM_TYPES).astype(pos.dtype)
    return out_a, out_b


# speedup vs baseline: 1.9087x; 1.0191x over previous
import functools

import jax
import jax.numpy as jnp
from jax import lax
from jax.experimental import pallas as pl
from jax.experimental.pallas import tpu as pltpu
from jax.experimental.pallas import tpu_sc as plsc

_NUM_TYPES = 64
_N = 100000
_LANES = 16
_GROUPS = _N // _LANES
_NC, _NS = 2, 16
_NW = _NC * _NS
_BASE_G = _GROUPS // _NW
_EXTRA = _GROUPS % _NW
_CG = 40
_NCHUNKS = 5
_CHUNK_ATOMS = _CG * _LANES
_ROW_W = _LANES * _NUM_TYPES
_CHUNK_WORDS = _CG * _ROW_W


def _one_hot_sc(types_flat):
    mesh = plsc.VectorSubcoreMesh(
        core_axis_name="c", subcore_axis_name="s",
        num_cores=_NC, num_subcores=_NS,
    )

    @functools.partial(
        pl.kernel,
        mesh=mesh,
        out_type=(
            jax.ShapeDtypeStruct((_N * _NUM_TYPES,), jnp.float32),
            jax.ShapeDtypeStruct((_N * _NUM_TYPES,), jnp.float32),
        ),
        scratch_types=[
            *[pltpu.VMEM((_CHUNK_ATOMS,), jnp.int32)
              for _ in range(_NCHUNKS)],
            pltpu.VMEM((_CHUNK_WORDS,), jnp.float32),
            pltpu.VMEM((_CHUNK_WORDS,), jnp.float32),
            pltpu.SemaphoreType.DMA,
            pltpu.SemaphoreType.DMA,
            pltpu.SemaphoreType.DMA,
        ],
        compiler_params=pltpu.CompilerParams(needs_layout_passes=False),
    )
    def k(types_hbm, out_a_hbm, out_b_hbm,
          i0, i1, i2, i3, i4, r0, r1, in_sem, osem0, osem1):
        idx = (i0, i1, i2, i3, i4)
        rows = (r0, r1)
        osem = (osem0, osem1)

        wid = lax.axis_index("s") * _NC + lax.axis_index("c")
        g0 = wid * _BASE_G + jnp.minimum(wid, _EXTRA)
        cnt = _BASE_G + jnp.where(wid < _EXTRA, 1, 0)
        gs = [g0 + c * _CG for c in range(_NCHUNKS - 1)]
        gs.append(g0 + cnt - _CG)

        in_descs = [
            pltpu.make_async_copy(
                types_hbm.at[pl.ds(gs[c] * _LANES, _CHUNK_ATOMS)],
                idx[c], in_sem)
            for c in range(_NCHUNKS)
        ]
        for d in in_descs:
            d.start()

        lane_row = lax.iota(jnp.int32, _LANES) * _NUM_TYPES
        ones = jnp.full((_LANES,), 1.0, jnp.float32)
        zeros = jnp.zeros((_LANES,), jnp.float32)

        def zbody(i, carry):
            base = i * (16 * _LANES)
            for u in range(16):
                r0[pl.ds(base + u * _LANES, _LANES)] = zeros
                r1[pl.ds(base + u * _LANES, _LANES)] = zeros
            return carry

        lax.fori_loop(0, _CHUNK_WORDS // (16 * _LANES), zbody, 0)

        for d in in_descs:
            d.wait()

        def scatter_pass(row_ref, idx_ref, val):
            def body(j, carry):
                t = idx_ref[pl.ds(j * _LANES, _LANES)]
                plsc.store_scatter(row_ref, [lane_row + t + j * _ROW_W],
                                   val)
                return carry
            lax.fori_loop(0, _CG, body, 0, unroll=8)

        out_descs = []
        for c in range(_NCHUNKS):
            b = c & 1
            if c >= 2:
                for d in out_descs[c - 2]:
                    d.wait()
                scatter_pass(rows[b], idx[c - 2], zeros)
            scatter_pass(rows[b], idx[c], ones)
            base = gs[c] * _ROW_W
            d_a = pltpu.make_async_copy(
                rows[b], out_a_hbm.at[pl.ds(base, _CHUNK_WORDS)], osem[b])
            d_b = pltpu.make_async_copy(
                rows[b], out_b_hbm.at[pl.ds(base, _CHUNK_WORDS)], osem[b])
            d_a.start()
            d_b.start()
            out_descs.append((d_a, d_b))

        for c in (_NCHUNKS - 2, _NCHUNKS - 1):
            for d in out_descs[c]:
                d.wait()

    return k(types_flat)


def kernel(atom_types, pos):
    out_a, out_b = _one_hot_sc(atom_types.reshape(-1))
    out_a = out_a.reshape(_N, _NUM_TYPES).astype(pos.dtype)
    out_b = out_b.reshape(_N, _NUM_TYPES).astype(pos.dtype)
    return out_a, out_b


# speedup vs baseline: 2.2942x; 1.2019x over previous
import functools

import jax
import jax.numpy as jnp
from jax import lax
from jax.experimental import pallas as pl
from jax.experimental.pallas import tpu as pltpu
from jax.experimental.pallas import tpu_sc as plsc

_NUM_TYPES = 64
_N = 100000
_LANES = 16
_GROUPS = _N // _LANES
_NC, _NS = 2, 16
_NW = _NC * _NS
_BASE_G = _GROUPS // _NW
_EXTRA = _GROUPS % _NW
_CG = 20
_NCHUNKS = 10
_CHUNK_ATOMS = _CG * _LANES
_ROW_W = _LANES * _NUM_TYPES
_CHUNK_WORDS = _CG * _ROW_W


def _one_hot_sc(types_flat):
    mesh = plsc.VectorSubcoreMesh(
        core_axis_name="c", subcore_axis_name="s",
        num_cores=_NC, num_subcores=_NS,
    )

    @functools.partial(
        pl.kernel,
        mesh=mesh,
        out_type=(
            jax.ShapeDtypeStruct((_N, _NUM_TYPES), jnp.float32),
            jax.ShapeDtypeStruct((_N, _NUM_TYPES), jnp.float32),
        ),
        scratch_types=[
            *[pltpu.VMEM((_CHUNK_ATOMS,), jnp.int32)
              for _ in range(_NCHUNKS)],
            pltpu.VMEM((_CHUNK_ATOMS, _NUM_TYPES), jnp.float32),
            pltpu.VMEM((_CHUNK_ATOMS, _NUM_TYPES), jnp.float32),
            pltpu.SemaphoreType.DMA,
            pltpu.SemaphoreType.DMA,
            pltpu.SemaphoreType.DMA,
        ],
        compiler_params=pltpu.CompilerParams(needs_layout_passes=False),
    )
    def k(types_hbm, out_a_hbm, out_b_hbm,
          i0, i1, i2, i3, i4, i5, i6, i7, i8, i9,
          r0, r1, in_sem, osem0, osem1):
        idx = (i0, i1, i2, i3, i4, i5, i6, i7, i8, i9)
        rows = (r0, r1)
        osem = (osem0, osem1)

        wid = lax.axis_index("s") * _NC + lax.axis_index("c")
        g0 = wid * _BASE_G + jnp.minimum(wid, _EXTRA)
        cnt = _BASE_G + jnp.where(wid < _EXTRA, 1, 0)
        gs = [g0 + c * _CG for c in range(_NCHUNKS - 1)]
        gs.append(g0 + cnt - _CG)

        in_descs = [
            pltpu.make_async_copy(
                types_hbm.at[pl.ds(gs[c] * _LANES, _CHUNK_ATOMS)],
                idx[c], in_sem)
            for c in range(_NCHUNKS)
        ]
        for d in in_descs:
            d.start()

        lane = lax.iota(jnp.int32, _LANES)
        ones = jnp.full((_LANES,), 1.0, jnp.float32)
        zeros = jnp.zeros((_LANES,), jnp.float32)

        def zbody(i, carry):
            rvec = jnp.full((_LANES,), 0, jnp.int32) + i
            for u in range(4):
                cvec = lane + u * _LANES
                plsc.store_scatter(r0, [rvec, cvec], zeros)
                plsc.store_scatter(r1, [rvec, cvec], zeros)
            return carry

        lax.fori_loop(0, _CHUNK_ATOMS, zbody, 0, unroll=8)

        for d in in_descs:
            d.wait()

        def scatter_pass(row_ref, idx_ref, val):
            def body(j, carry):
                t = idx_ref[pl.ds(j * _LANES, _LANES)]
                plsc.store_scatter(row_ref, [lane + j * _LANES, t], val)
                return carry
            lax.fori_loop(0, _CG, body, 0, unroll=8)

        out_descs = []
        for c in range(_NCHUNKS):
            b = c & 1
            if c >= 2:
                for d in out_descs[c - 2]:
                    d.wait()
                scatter_pass(rows[b], idx[c - 2], zeros)
            scatter_pass(rows[b], idx[c], ones)
            base = gs[c] * _LANES
            d_a = pltpu.make_async_copy(
                rows[b], out_a_hbm.at[pl.ds(base, _CHUNK_ATOMS)], osem[b])
            d_b = pltpu.make_async_copy(
                rows[b], out_b_hbm.at[pl.ds(base, _CHUNK_ATOMS)], osem[b])
            d_a.start()
            d_b.start()
            out_descs.append((d_a, d_b))

        for c in (_NCHUNKS - 2, _NCHUNKS - 1):
            for d in out_descs[c]:
                d.wait()

    return k(types_flat)


def kernel(atom_types, pos):
    out_a, out_b = _one_hot_sc(atom_types.reshape(-1))
    return out_a.astype(pos.dtype), out_b.astype(pos.dtype)


# speedup vs baseline: 2.2963x; 1.0009x over previous
import functools

import jax
import jax.numpy as jnp
from jax import lax
from jax.experimental import pallas as pl
from jax.experimental.pallas import tpu as pltpu
from jax.experimental.pallas import tpu_sc as plsc

_NUM_TYPES = 64
_N = 100000
_LANES = 16
_GROUPS = _N // _LANES
_NC, _NS = 2, 16
_NW = _NC * _NS
_BASE_G = _GROUPS // _NW
_EXTRA = _GROUPS % _NW
_CG = 20
_NCHUNKS = 10
_CHUNK_ATOMS = _CG * _LANES
_ROW_W = _LANES * _NUM_TYPES
_CHUNK_WORDS = _CG * _ROW_W


def _one_hot_sc(types_flat):
    mesh = plsc.VectorSubcoreMesh(
        core_axis_name="c", subcore_axis_name="s",
        num_cores=_NC, num_subcores=_NS,
    )

    @functools.partial(
        pl.kernel,
        mesh=mesh,
        out_type=(
            jax.ShapeDtypeStruct((_N, _NUM_TYPES), jnp.float32),
            jax.ShapeDtypeStruct((_N, _NUM_TYPES), jnp.float32),
        ),
        scratch_types=[
            *[pltpu.VMEM((_CHUNK_ATOMS,), jnp.int32)
              for _ in range(_NCHUNKS)],
            pltpu.VMEM((_CHUNK_ATOMS, _NUM_TYPES), jnp.float32),
            pltpu.VMEM((_CHUNK_ATOMS, _NUM_TYPES), jnp.float32),
            pltpu.SemaphoreType.DMA,
            pltpu.SemaphoreType.DMA,
            pltpu.SemaphoreType.DMA,
        ],
        compiler_params=pltpu.CompilerParams(needs_layout_passes=False,
                                             use_tc_tiling_on_sc=True),
    )
    def k(types_hbm, out_a_hbm, out_b_hbm,
          i0, i1, i2, i3, i4, i5, i6, i7, i8, i9,
          r0, r1, in_sem, osem0, osem1):
        idx = (i0, i1, i2, i3, i4, i5, i6, i7, i8, i9)
        rows = (r0, r1)
        osem = (osem0, osem1)

        wid = lax.axis_index("s") * _NC + lax.axis_index("c")
        g0 = wid * _BASE_G + jnp.minimum(wid, _EXTRA)
        cnt = _BASE_G + jnp.where(wid < _EXTRA, 1, 0)
        gs = [g0 + c * _CG for c in range(_NCHUNKS - 1)]
        gs.append(g0 + cnt - _CG)

        in_descs = [
            pltpu.make_async_copy(
                types_hbm.at[pl.ds(gs[c] * _LANES, _CHUNK_ATOMS)],
                idx[c], in_sem)
            for c in range(_NCHUNKS)
        ]
        for d in in_descs:
            d.start()

        lane = lax.iota(jnp.int32, _LANES)
        ones = jnp.full((_LANES,), 1.0, jnp.float32)
        zeros = jnp.zeros((_LANES,), jnp.float32)

        def zbody(i, carry):
            rvec = jnp.full((_LANES,), 0, jnp.int32) + i
            for u in range(4):
                cvec = lane + u * _LANES
                plsc.store_scatter(r0, [rvec, cvec], zeros)
                plsc.store_scatter(r1, [rvec, cvec], zeros)
            return carry

        lax.fori_loop(0, _CHUNK_ATOMS, zbody, 0, unroll=8)

        for d in in_descs:
            d.wait()

        def scatter_pass(row_ref, idx_ref, val):
            def body(j, carry):
                t = idx_ref[pl.ds(j * _LANES, _LANES)]
                plsc.store_scatter(row_ref, [lane + j * _LANES, t], val)
                return carry
            lax.fori_loop(0, _CG, body, 0, unroll=8)

        out_descs = []
        for c in range(_NCHUNKS):
            b = c & 1
            if c >= 2:
                for d in out_descs[c - 2]:
                    d.wait()
                scatter_pass(rows[b], idx[c - 2], zeros)
            scatter_pass(rows[b], idx[c], ones)
            base = gs[c] * _LANES
            d_a = pltpu.make_async_copy(
                rows[b], out_a_hbm.at[pl.ds(base, _CHUNK_ATOMS)], osem[b])
            d_b = pltpu.make_async_copy(
                rows[b], out_b_hbm.at[pl.ds(base, _CHUNK_ATOMS)], osem[b])
            d_a.start()
            d_b.start()
            out_descs.append((d_a, d_b))

        for c in (_NCHUNKS - 2, _NCHUNKS - 1):
            for d in out_descs[c]:
                d.wait()

    return k(types_flat)


def kernel(atom_types, pos):
    out_a, out_b = _one_hot_sc(atom_types.reshape(-1))
    return out_a.astype(pos.dtype), out_b.astype(pos.dtype)


# speedup vs baseline: 7.0538x; 3.0718x over previous
import functools

import jax
import jax.numpy as jnp
from jax import lax
from jax.experimental import pallas as pl
from jax.experimental.pallas import tpu as pltpu
from jax.experimental.pallas import tpu_sc as plsc

_NUM_TYPES = 64
_N = 100000
_LANES = 16
_NC, _NS = 2, 16
_NW = _NC * _NS
_CW = 128
_FULL_CHUNKS = _N // _CW
_TAIL = _N - _FULL_CHUNKS * _CW
_SLOTS = -(-_FULL_CHUNKS // _NW)
_GP = _CW // _LANES
_TAIL_GP = _TAIL // _LANES
_TAIL_W = _NW - 1


def _one_hot_t_sc(types_flat):
    mesh = plsc.VectorSubcoreMesh(
        core_axis_name="c", subcore_axis_name="s",
        num_cores=_NC, num_subcores=_NS,
    )

    @functools.partial(
        pl.kernel,
        mesh=mesh,
        out_type=(
            jax.ShapeDtypeStruct((_NUM_TYPES, _N), jnp.float32),
            jax.ShapeDtypeStruct((_NUM_TYPES, _N), jnp.float32),
        ),
        scratch_types=[
            pltpu.VMEM((_SLOTS * _CW,), jnp.int32),
            pltpu.VMEM((_TAIL,), jnp.int32),
            pltpu.VMEM((_NUM_TYPES, _CW), jnp.float32),
            pltpu.VMEM((_NUM_TYPES, _CW), jnp.float32),
            pltpu.VMEM((_NUM_TYPES, _TAIL), jnp.float32),
            pltpu.SemaphoreType.DMA,
            pltpu.SemaphoreType.DMA,
            pltpu.SemaphoreType.DMA,
            pltpu.SemaphoreType.DMA,
        ],
        compiler_params=pltpu.CompilerParams(needs_layout_passes=False,
                                             use_tc_tiling_on_sc=True),
    )
    def k(types_hbm, out_a_hbm, out_b_hbm,
          idx_all, idx_tail, r0, r1, r_tail,
          in_sem, osem0, osem1, tsem):
        rows = (r0, r1)
        osem = (osem0, osem1)

        wid = lax.axis_index("s") * _NC + lax.axis_index("c")

        ks = []
        for s in range(_SLOTS):
            k_s = wid + _NW * s
            if (_NW - 1) + _NW * s >= _FULL_CHUNKS:
                k_s = jnp.where(k_s < _FULL_CHUNKS, k_s, wid)
            ks.append(pl.multiple_of(k_s * _CW, _CW))

        in_descs = [
            pltpu.make_async_copy(
                types_hbm.at[pl.ds(ks[s], _CW)],
                idx_all.at[pl.ds(s * _CW, _CW)], in_sem)
            for s in range(_SLOTS)
        ]
        in_descs.append(pltpu.make_async_copy(
            types_hbm.at[pl.ds(_FULL_CHUNKS * _CW, _TAIL)], idx_tail,
            in_sem))
        for d in in_descs:
            d.start()

        lane = lax.iota(jnp.int32, _LANES)
        ones = jnp.full((_LANES,), 1.0, jnp.float32)
        zeros = jnp.zeros((_LANES,), jnp.float32)

        def zbody(i, carry):
            rvec = jnp.full((_LANES,), 0, jnp.int32) + i
            for u in range(_CW // _LANES):
                cvec = lane + u * _LANES
                plsc.store_scatter(r0, [rvec, cvec], zeros)
                plsc.store_scatter(r1, [rvec, cvec], zeros)
            for u in range(_TAIL // _LANES):
                cvec = lane + u * _LANES
                plsc.store_scatter(r_tail, [rvec, cvec], zeros)
            return carry

        lax.fori_loop(0, _NUM_TYPES, zbody, 0, unroll=4)

        for d in in_descs:
            d.wait()

        def scatter_pass(row_ref, idx_base, ngroups, val):
            for j in range(ngroups):
                t = idx_all[pl.ds(idx_base + j * _LANES, _LANES)]
                plsc.store_scatter(row_ref, [t, lane + j * _LANES], val)

        out_descs = []
        for s in range(_SLOTS):
            b = s & 1
            if s >= 2:
                for d in out_descs[s - 2]:
                    d.wait()
                scatter_pass(rows[b], (s - 2) * _CW, _GP, zeros)
            scatter_pass(rows[b], s * _CW, _GP, ones)
            d_a = pltpu.make_async_copy(
                rows[b], out_a_hbm.at[:, pl.ds(ks[s], _CW)], osem[b])
            d_b = pltpu.make_async_copy(
                rows[b], out_b_hbm.at[:, pl.ds(ks[s], _CW)], osem[b])
            d_a.start()
            d_b.start()
            out_descs.append((d_a, d_b))

        tail_descs = [
            pltpu.make_async_copy(
                r_tail,
                out_a_hbm.at[:, pl.ds(_FULL_CHUNKS * _CW, _TAIL)], tsem),
            pltpu.make_async_copy(
                r_tail,
                out_b_hbm.at[:, pl.ds(_FULL_CHUNKS * _CW, _TAIL)], tsem),
        ]

        @pl.when(wid == _TAIL_W)
        def _():
            for j in range(_TAIL_GP):
                t = idx_tail[pl.ds(j * _LANES, _LANES)]
                plsc.store_scatter(r_tail, [t, lane + j * _LANES], ones)
            for d in tail_descs:
                d.start()
            for d in tail_descs:
                d.wait()

        for s in (_SLOTS - 2, _SLOTS - 1):
            for d in out_descs[s]:
                d.wait()

    return k(types_flat)


def kernel(atom_types, pos):
    out_a_t, out_b_t = _one_hot_t_sc(atom_types.reshape(-1))
    return out_a_t.T.astype(pos.dtype), out_b_t.T.astype(pos.dtype)
